# baseline (speedup 1.0000x reference)
import functools

import jax
import jax.numpy as jnp
from jax import lax
from jax.experimental import pallas as pl
from jax.experimental.pallas import tpu as pltpu
from jax.experimental.pallas import tpu_sc as plsc

D_MODEL = 1024
BATCH = 4
SEQ = 4096
B_TOTAL = BATCH * SEQ
NUM_CORES = 2
NUM_SUBCORES = 16
NW = NUM_CORES * NUM_SUBCORES
B_PER_W = B_TOTAL // NW
W_PER_BATCH = NW // BATCH
CHUNK = 16
NCHUNK = B_PER_W // CHUNK
NBUF = 7


def _pe_gather(x, pe):
    mesh = plsc.VectorSubcoreMesh(core_axis_name="c", subcore_axis_name="s")

    @functools.partial(
        pl.kernel,
        mesh=mesh,
        out_type=jax.ShapeDtypeStruct((B_TOTAL, D_MODEL), jnp.float32),
        scratch_types=[
            pltpu.VMEM((B_PER_W,), jnp.int32),
        ]
        + [pltpu.VMEM((CHUNK, D_MODEL), jnp.float32) for _ in range(NBUF)]
        + [pltpu.SemaphoreType.DMA for _ in range(2 * NBUF)],
    )
    def k(idx_hbm, table_hbm, out_hbm, idx_v, *scratch):
        bufs = scratch[:NBUF]
        gsems = scratch[NBUF:2 * NBUF]
        osems = scratch[2 * NBUF:]
        wid = lax.axis_index("s") * NUM_CORES + lax.axis_index("c")
        batch = wid // W_PER_BATCH
        col0 = (wid % W_PER_BATCH) * B_PER_W
        base = wid * B_PER_W
        pltpu.sync_copy(idx_hbm.at[batch, pl.ds(col0, B_PER_W)], idx_v)
        gcp = [None] * NBUF
        ocp = [None] * NBUF
        for g in range(NBUF):
            gcp[g] = pltpu.async_copy(
                table_hbm.at[idx_v.at[pl.ds(g * CHUNK, CHUNK)]],
                bufs[g], gsems[g])
        for c in range(NCHUNK):
            b = c % NBUF
            gcp[b].wait()
            ocp[b] = pltpu.async_copy(
                bufs[b], out_hbm.at[pl.ds(base + c * CHUNK, CHUNK)],
                osems[b])
            g = c + NBUF
            if g < NCHUNK:
                ocp[b].wait()
                gcp[b] = pltpu.async_copy(
                    table_hbm.at[idx_v.at[pl.ds(g * CHUNK, CHUNK)]],
                    bufs[b], gsems[b])
        for c in range(NCHUNK - NBUF, NCHUNK):
            ocp[c % NBUF].wait()

    return k(x, pe)


def kernel(x, pe):
    if x.dtype != jnp.int32:
        x = x.astype(jnp.int32)
    if pe.dtype != jnp.float32:
        pe = pe.astype(jnp.float32)
    out = _pe_gather(x, pe)
    return out.reshape(x.shape + (D_MODEL,))


# speedup vs baseline: 1.0115x; 1.0115x over previous
import functools

import jax
import jax.numpy as jnp
from jax import lax
from jax.experimental import pallas as pl
from jax.experimental.pallas import tpu as pltpu
from jax.experimental.pallas import tpu_sc as plsc

D_MODEL = 1024
BATCH = 4
SEQ = 4096
B_TOTAL = BATCH * SEQ
NUM_CORES = 2
NUM_SUBCORES = 16
NW = NUM_CORES * NUM_SUBCORES
B_PER_W = B_TOTAL // NW
W_PER_BATCH = NW // BATCH
CHUNK = 16
NCHUNK = B_PER_W // CHUNK
NBUF = 6


def _pe_gather(x, pe):
    mesh = plsc.VectorSubcoreMesh(core_axis_name="c", subcore_axis_name="s")

    @functools.partial(
        pl.kernel,
        mesh=mesh,
        out_type=jax.ShapeDtypeStruct((B_TOTAL, D_MODEL), jnp.float32),
        scratch_types=[
            pltpu.VMEM((B_PER_W,), jnp.int32),
        ]
        + [pltpu.VMEM((CHUNK, D_MODEL), jnp.float32) for _ in range(NBUF)]
        + [pltpu.SemaphoreType.DMA for _ in range(2 * NBUF)],
    )
    def k(idx_hbm, table_hbm, out_hbm, idx_v, *scratch):
        bufs = scratch[:NBUF]
        gsems = scratch[NBUF:2 * NBUF]
        osems = scratch[2 * NBUF:]
        wid = lax.axis_index("s") * NUM_CORES + lax.axis_index("c")
        batch = wid // W_PER_BATCH
        col0 = (wid % W_PER_BATCH) * B_PER_W
        base = wid * B_PER_W
        pltpu.sync_copy(idx_hbm.at[batch, pl.ds(col0, B_PER_W)], idx_v)
        gcp = [None] * NBUF
        ocp = [None] * NBUF
        for g in range(NBUF):
            gcp[g] = pltpu.async_copy(
                table_hbm.at[idx_v.at[pl.ds(g * CHUNK, CHUNK)]],
                bufs[g], gsems[g])
        for c in range(NCHUNK):
            b = c % NBUF
            gcp[b].wait()
            ocp[b] = pltpu.async_copy(
                bufs[b], out_hbm.at[pl.ds(base + c * CHUNK, CHUNK)],
                osems[b])
            g = c + NBUF
            if g < NCHUNK:
                ocp[b].wait()
                gcp[b] = pltpu.async_copy(
                    table_hbm.at[idx_v.at[pl.ds(g * CHUNK, CHUNK)]],
                    bufs[b], gsems[b])
        for c in range(NCHUNK - NBUF, NCHUNK):
            ocp[c % NBUF].wait()

    return k(x, pe)


def kernel(x, pe):
    if x.dtype != jnp.int32:
        x = x.astype(jnp.int32)
    if pe.dtype != jnp.float32:
        pe = pe.astype(jnp.float32)
    out = _pe_gather(x, pe)
    return out.reshape(x.shape + (D_MODEL,))
